# baseline (speedup 1.0000x reference)
import functools

import jax
import jax.numpy as jnp
import numpy as np
from jax import lax
from jax.experimental import pallas as pl
from jax.experimental.pallas import tpu as pltpu
from jax.experimental.pallas import tpu_sc as plsc

NEG = -1e9
BIGI = 2**30
IMIN = -2**31
MASK31 = 0x7FFFFFFF
CONST_NO = 100000
KF = 64
KR = 16
GQ = 8
NROWPAD = 4096


def _sortkey(x):
    b = jax.lax.bitcast_convert_type(x, jnp.int32)
    return jnp.where(b >= 0, b, b ^ MASK31)


KEY_NEG1E8 = int(np.int32(np.float32(-1e8).view(np.int32)) ^ MASK31)
KEY_NEG = int(np.int32(np.float32(NEG).view(np.int32)) ^ MASK31)


def _iota(shape, dim):
    return jax.lax.broadcasted_iota(jnp.int32, shape, dim)


def _pass1_body(nrow, r_count, bmax,
                qarr_ref, rlens_ref,
                fpred_ref, fa0_ref, fa1_ref, wts_ref,
                fpredT_ref, fa0T_ref, fa1T_ref, wtsT_ref,
                rhead_ref, bodies_ref,
                keys_ref, rowmax_ref, mbodies_ref, mlens_ref,
                rsc_ref):
    g = pl.program_id(0)
    iota_r = _iota((1, r_count), 1)
    riota_f = iota_r.astype(jnp.float32) * jnp.float32(1e-6)

    wkey = _sortkey(wts_ref[...])
    wkeyT = _sortkey(wtsT_ref[...])
    actives = []
    for qi in range(GQ):
        qp = qarr_ref[g * GQ + qi, 0]
        q0 = qarr_ref[g * GQ + qi, 1]
        q1 = qarr_ref[g * GQ + qi, 2]
        actives.append(qp != 0)
        match = (fpred_ref[...] == qp)
        match &= (q0 > CONST_NO) | (fa0_ref[...] == q0)
        match &= (q1 > CONST_NO) | (fa1_ref[...] == q1)
        keys_ref[qi] = jnp.where(match, wkey, KEY_NEG)
        matchT = (fpredT_ref[...] == qp)
        matchT &= (q0 > CONST_NO) | (fa0T_ref[...] == q0)
        matchT &= (q1 > CONST_NO) | (fa1T_ref[...] == q1)
        scT = jnp.where(matchT, wkeyT, KEY_NEG)
        rowmax_ref[qi:qi + 1, 0:nrow] = jnp.max(scT, axis=0).reshape(1, nrow)
        rh = rhead_ref[...] == qp
        rsc_ref[qi:qi + 1, :] = jnp.where(rh, jnp.float32(1.0), NEG) - riota_f
    if nrow < NROWPAD:
        rowmax_ref[:, nrow:] = jnp.full((GQ, NROWPAD - nrow), IMIN, jnp.int32)

    def rule_body(t, carry):
        mb, ml = carry
        for qi in range(GQ):
            rs = rsc_ref[qi:qi + 1, :]
            m = jnp.max(rs)
            r = jnp.min(jnp.where(rs == m, iota_r, BIGI))
            rsc_ref[qi:qi + 1, :] = jnp.where(iota_r == r, -3.0e38, rs)
            validr = (m > jnp.float32(-1e8)) & actives[qi]
            brow = bodies_ref[pl.ds(r, 1), :]
            brow = jnp.where(validr, brow, 0)
            lv = jnp.where(validr, rlens_ref[0, r], 0)
            mask3 = ((_iota((GQ, KR, 1), 0) == qi)
                     & (_iota((GQ, KR, 1), 1) == t))
            mb = jnp.where(mask3, brow[None], mb)
            sel = (_iota((GQ, KR), 0) == qi) & (_iota((GQ, KR), 1) == t)
            ml = jnp.where(sel, lv, ml)
        return mb, ml

    mb, ml = jax.lax.fori_loop(
        0, KR, rule_body,
        (jnp.zeros((GQ, KR, 3 * bmax), jnp.int32),
         jnp.zeros((GQ, KR), jnp.int32)))
    mbodies_ref[...] = mb
    mlens_ref[...] = ml


def _sc_body(keys_hbm, rowmax_hbm, qpred_hbm, fpred_hbm, fa0_hbm, fa1_hbm,
             vout_hbm, mpred_hbm, ma0_hbm, ma1_hbm,
             rowmax_v, lvl2_v, rowbuf, idxbuf, keybuf, obuf,
             gpred, ga0, ga1, qv_v, dirty_v, dmap_v, sem):
    q = lax.axis_index("s") * 2 + lax.axis_index("c")
    iota16 = jax.lax.broadcasted_iota(jnp.int32, (16,), 0)

    pltpu.sync_copy(rowmax_hbm.at[q], rowmax_v)
    pltpu.sync_copy(qpred_hbm, qv_v)

    l3v = jnp.full((16,), IMIN, jnp.int32)
    for k in range(16):
        acc = jnp.full((16,), IMIN, jnp.int32)
        for o in range(16):
            v = rowmax_v[pl.ds((k * 16 + o) * 16, 16)]
            acc = jnp.where(iota16 == o, jnp.max(v), acc)
        lvl2_v[pl.ds(k * 16, 16)] = acc
        l3v = jnp.where(iota16 == k, jnp.max(acc), l3v)

    for j in range(4):
        dmap_v[pl.ds(j * 16, 16)] = jnp.full((16,), -1, jnp.int32)

    def ext_body(t, carry):
        l3v, nslots = carry
        m = jnp.max(l3v)
        kstar = jnp.min(jnp.where(l3v == m, iota16, 16))
        lv2k = lvl2_v[pl.ds(kstar * 16, 16)]
        tstar = jnp.min(jnp.where(lv2k == m, iota16, 16))
        cb = kstar * 256 + tstar * 16
        ch = rowmax_v[pl.ds(cb, 16)]
        lstar = jnp.min(jnp.where(ch == m, iota16, 16))
        r = cb + lstar

        slotvec = jnp.full((16,), BIGI, jnp.int32)
        for j in range(4):
            mv = dmap_v[pl.ds(j * 16, 16)]
            slotvec = jnp.minimum(
                slotvec, jnp.where(mv == r, iota16 + j * 16, BIGI))
        found_slot = jnp.min(slotvec)
        hit = found_slot < BIGI
        slot = jnp.where(hit, found_slot, nslots)
        hitv = iota16 * 0 + jnp.where(hit, 1, 0) > 0

        pltpu.sync_copy(keys_hbm.at[q, r], rowbuf)
        jb = (nslots // 16) * 16
        mv = dmap_v[pl.ds(jb, 16)]
        dmap_v[pl.ds(jb, 16)] = jnp.where(
            (iota16 == nslots - jb) & jnp.logical_not(hitv), r, mv)
        nslots = jnp.where(hit, nslots, nslots + 1)

        rowvs = []
        minvec = jnp.full((16,), BIGI, jnp.int32)
        for j in range(8):
            fresh = rowbuf[pl.ds(j * 16, 16)]
            cached = dirty_v[pl.ds(slot * 128 + j * 16, 16)]
            v = jnp.where(hitv, cached, fresh)
            rowvs.append(v)
            minvec = jnp.minimum(
                minvec, jnp.where(v == m, iota16 + j * 16, BIGI))
        c = jnp.min(minvec)
        nmvec = jnp.full((16,), IMIN, jnp.int32)
        for j in range(8):
            v = jnp.where(iota16 + j * 16 == c, IMIN, rowvs[j])
            dirty_v[pl.ds(slot * 128 + j * 16, 16)] = v
            nmvec = jnp.maximum(nmvec, v)
        nm = jnp.max(nmvec)
        gidx = r * 128 + c

        tb = (t // 16) * 16
        iv = idxbuf[pl.ds(tb, 16)]
        idxbuf[pl.ds(tb, 16)] = jnp.where(iota16 == t - tb, gidx, iv)
        kv = keybuf[pl.ds(tb, 16)]
        keybuf[pl.ds(tb, 16)] = jnp.where(iota16 == t - tb, m, kv)
        rowmax_v[pl.ds(cb, 16)] = jnp.where(iota16 == lstar, nm, ch)
        nm2 = jnp.max(jnp.where(iota16 == lstar, nm, ch))
        lvl2_v[pl.ds(kstar * 16, 16)] = jnp.where(
            iota16 == tstar, nm2, lv2k)
        lv2k2 = jnp.where(iota16 == tstar, nm2, lv2k)
        return (jnp.where(iota16 == kstar, jnp.max(lv2k2), l3v),
                nslots)

    lax.fori_loop(0, KF, ext_body, (l3v, jnp.int32(0)))

    pltpu.async_copy(fpred_hbm.at[idxbuf], gpred, sem).wait()
    pltpu.async_copy(fa0_hbm.at[idxbuf], ga0, sem).wait()
    pltpu.async_copy(fa1_hbm.at[idxbuf], ga1, sem).wait()

    qb = (q // 16) * 16
    qchunk = qv_v[pl.ds(qb, 16)]
    qsel = jnp.where(iota16 == q - qb, qchunk, 0)
    act = iota16 * 0 + jnp.max(qsel) != 0
    for j in range(4):
        sl = pl.ds(j * 16, 16)
        kb = keybuf[sl]
        valid = (kb > KEY_NEG1E8) & act
        fb = plsc.bitcast(jnp.where(kb >= 0, kb, kb ^ MASK31), jnp.float32)
        obuf[sl] = jnp.where(valid, fb, jnp.float32(0.0))
        gpred[sl] = jnp.where(valid, gpred[sl], 0)
        ga0[sl] = jnp.where(valid, ga0[sl], 0)
        ga1[sl] = jnp.where(valid, ga1[sl], 0)

    pltpu.sync_copy(obuf, vout_hbm.at[q])
    pltpu.sync_copy(gpred, mpred_hbm.at[q])
    pltpu.sync_copy(ga0, ma0_hbm.at[q])
    pltpu.sync_copy(ga1, ma1_hbm.at[q])


def _pred_body(vout_ref, qpred_ref, pred_ref):
    w = jnp.sum(vout_ref[...], axis=1, keepdims=True)
    for ci in range(2):
        classes = _iota((1, 128), 1) + ci * 128
        oh = qpred_ref[...] == classes
        pred_ref[0:1, pl.ds(ci * 128, 128)] = jnp.sum(
            jnp.where(oh, w, jnp.float32(0.0)), axis=0).reshape(1, 128)


def kernel(proof_goals, facts_idx, rules_heads_idx, rules_bodies_idx,
           rule_lens, fact_weights):
    B, S, G, _ = proof_goals.shape
    F = facts_idx.shape[0]
    R = rules_bodies_idx.shape[0]
    BMAX = rules_bodies_idx.shape[1]
    nq = B * S
    nrow = pl.cdiv(F, 128)
    nrow = ((nrow + 127) // 128) * 128
    fp = nrow * 128

    qarr = proof_goals[:, :, 0, :].reshape(nq, 3)
    pad = fp - F
    fpred = jnp.concatenate([facts_idx[:, 0], jnp.zeros((pad,), facts_idx.dtype)]).reshape(nrow, 128)
    fa0 = jnp.concatenate([facts_idx[:, 1], jnp.zeros((pad,), facts_idx.dtype)]).reshape(nrow, 128)
    fa1 = jnp.concatenate([facts_idx[:, 2], jnp.zeros((pad,), facts_idx.dtype)]).reshape(nrow, 128)
    wts = jnp.concatenate([fact_weights, jnp.zeros((pad,), fact_weights.dtype)]).reshape(nrow, 128)
    rhead = rules_heads_idx[:, 0].reshape(1, R)
    bodies = rules_bodies_idx.reshape(R, 3 * BMAX)
    rlens = rule_lens.reshape(1, R)

    ngroups = nq // GQ
    smem = pl.BlockSpec(memory_space=pltpu.SMEM)
    vfull = pl.BlockSpec(memory_space=pltpu.VMEM)

    keys, rowmax, mbodies, mlens = pl.pallas_call(
        functools.partial(_pass1_body, nrow, R, BMAX),
        grid=(ngroups,),
        out_shape=(
            jax.ShapeDtypeStruct((nq, nrow, 128), jnp.int32),
            jax.ShapeDtypeStruct((nq, NROWPAD), jnp.int32),
            jax.ShapeDtypeStruct((nq, KR, 3 * BMAX), jnp.int32),
            jax.ShapeDtypeStruct((nq, KR), jnp.int32),
        ),
        in_specs=[
            pl.BlockSpec((nq, 3), lambda i: (0, 0), memory_space=pltpu.SMEM),
            pl.BlockSpec((1, R), lambda i: (0, 0), memory_space=pltpu.SMEM),
            pl.BlockSpec((nrow, 128), lambda i: (0, 0)),
            pl.BlockSpec((nrow, 128), lambda i: (0, 0)),
            pl.BlockSpec((nrow, 128), lambda i: (0, 0)),
            pl.BlockSpec((nrow, 128), lambda i: (0, 0)),
            pl.BlockSpec((128, nrow), lambda i: (0, 0)),
            pl.BlockSpec((128, nrow), lambda i: (0, 0)),
            pl.BlockSpec((128, nrow), lambda i: (0, 0)),
            pl.BlockSpec((128, nrow), lambda i: (0, 0)),
            pl.BlockSpec((1, R), lambda i: (0, 0)),
            pl.BlockSpec((R, 3 * BMAX), lambda i: (0, 0)),
        ],
        out_specs=(
            pl.BlockSpec((GQ, nrow, 128), lambda i: (i, 0, 0)),
            pl.BlockSpec((GQ, NROWPAD), lambda i: (i, 0)),
            pl.BlockSpec((GQ, KR, 3 * BMAX), lambda i: (i, 0, 0)),
            pl.BlockSpec((GQ, KR), lambda i: (i, 0)),
        ),
        scratch_shapes=[pltpu.VMEM((GQ, R), jnp.float32)],
    )(qarr, rlens, fpred, fa0, fa1, wts,
      fpred.T, fa0.T, fa1.T, wts.T, rhead, bodies)

    mesh = plsc.VectorSubcoreMesh(core_axis_name="c", subcore_axis_name="s")
    sc_out = (
        jax.ShapeDtypeStruct((nq, KF), jnp.float32),
        jax.ShapeDtypeStruct((nq, KF), jnp.int32),
        jax.ShapeDtypeStruct((nq, KF), jnp.int32),
        jax.ShapeDtypeStruct((nq, KF), jnp.int32),
    )
    sc_scratch = [
        pltpu.VMEM((NROWPAD,), jnp.int32),
        pltpu.VMEM((256,), jnp.int32),
        pltpu.VMEM((128,), jnp.int32),
        pltpu.VMEM((KF,), jnp.int32),
        pltpu.VMEM((KF,), jnp.int32),
        pltpu.VMEM((KF,), jnp.float32),
        pltpu.VMEM((KF,), jnp.int32),
        pltpu.VMEM((KF,), jnp.int32),
        pltpu.VMEM((KF,), jnp.int32),
        pltpu.VMEM((nq,), jnp.int32),
        pltpu.VMEM((KF * 128,), jnp.int32),
        pltpu.VMEM((KF,), jnp.int32),
        pltpu.SemaphoreType.DMA,
    ]
    qpredv = qarr[:, 0]
    vout, mpred, ma0, ma1 = pl.kernel(
        _sc_body, mesh=mesh, out_type=sc_out, scratch_types=sc_scratch,
        compiler_params=pltpu.CompilerParams(needs_layout_passes=False),
    )(keys, rowmax, qpredv, fpred.reshape(fp), fa0.reshape(fp),
      fa1.reshape(fp))

    pred = pl.pallas_call(
        _pred_body,
        out_shape=jax.ShapeDtypeStruct((1, 256), jnp.float32),
        in_specs=[vfull, vfull],
        out_specs=vfull,
    )(vout, qpredv.reshape(nq, 1))

    matched_facts = jnp.stack([mpred, ma0, ma1], axis=-1).reshape(B, S, KF, 3)
    return (vout.reshape(B, S, KF),
            matched_facts,
            mbodies.reshape(B, S, KR, BMAX, 3),
            mlens.reshape(B, S, KR),
            pred[0, :201])


# speedup vs baseline: 1.0587x; 1.0587x over previous
import functools

import jax
import jax.numpy as jnp
import numpy as np
from jax import lax
from jax.experimental import pallas as pl
from jax.experimental.pallas import tpu as pltpu
from jax.experimental.pallas import tpu_sc as plsc

NEG = -1e9
BIGI = 2**30
IMIN = -2**31
MASK31 = 0x7FFFFFFF
CONST_NO = 100000
KF = 64
KR = 16
GQ = 8
NROWPAD = 4096


def _sortkey(x):
    b = jax.lax.bitcast_convert_type(x, jnp.int32)
    return jnp.where(b >= 0, b, b ^ MASK31)


KEY_NEG1E8 = int(np.int32(np.float32(-1e8).view(np.int32)) ^ MASK31)
KEY_NEG = int(np.int32(np.float32(NEG).view(np.int32)) ^ MASK31)


def _iota(shape, dim):
    return jax.lax.broadcasted_iota(jnp.int32, shape, dim)


def _pass1_body(nrow, r_count, bmax,
                qarr_ref, rlens_ref,
                fpred_ref, fa0_ref, fa1_ref, wts_ref, rhead_ref, bodies_ref,
                keys_ref, rowmax_ref, mbodies_ref, mlens_ref,
                rsc_ref):
    g = pl.program_id(0)
    iota_r = _iota((1, r_count), 1)
    riota_f = iota_r.astype(jnp.float32) * jnp.float32(1e-6)

    wkey = _sortkey(wts_ref[...])
    actives = []
    for qi in range(GQ):
        qp = qarr_ref[g * GQ + qi, 0]
        q0 = qarr_ref[g * GQ + qi, 1]
        q1 = qarr_ref[g * GQ + qi, 2]
        actives.append(qp != 0)
        match = (fpred_ref[...] == qp)
        match &= (q0 > CONST_NO) | (fa0_ref[...] == q0)
        match &= (q1 > CONST_NO) | (fa1_ref[...] == q1)
        sc = jnp.where(match, wkey, KEY_NEG)
        keys_ref[qi] = sc
        rm = jnp.max(sc, axis=1)
        rowmax_ref[qi:qi + 1, 0:nrow] = rm.reshape(1, nrow)
        rh = rhead_ref[...] == qp
        rsc_ref[qi:qi + 1, :] = jnp.where(rh, jnp.float32(1.0), NEG) - riota_f
    if nrow < NROWPAD:
        rowmax_ref[:, nrow:] = jnp.full((GQ, NROWPAD - nrow), IMIN, jnp.int32)

    def rule_body(t, carry):
        mb, ml = carry
        for qi in range(GQ):
            rs = rsc_ref[qi:qi + 1, :]
            m = jnp.max(rs)
            r = jnp.min(jnp.where(rs == m, iota_r, BIGI))
            rsc_ref[qi:qi + 1, :] = jnp.where(iota_r == r, -3.0e38, rs)
            validr = (m > jnp.float32(-1e8)) & actives[qi]
            brow = bodies_ref[pl.ds(r, 1), :]
            brow = jnp.where(validr, brow, 0)
            lv = jnp.where(validr, rlens_ref[0, r], 0)
            mask3 = ((_iota((GQ, KR, 1), 0) == qi)
                     & (_iota((GQ, KR, 1), 1) == t))
            mb = jnp.where(mask3, brow[None], mb)
            sel = (_iota((GQ, KR), 0) == qi) & (_iota((GQ, KR), 1) == t)
            ml = jnp.where(sel, lv, ml)
        return mb, ml

    mb, ml = jax.lax.fori_loop(
        0, KR, rule_body,
        (jnp.zeros((GQ, KR, 3 * bmax), jnp.int32),
         jnp.zeros((GQ, KR), jnp.int32)))
    mbodies_ref[...] = mb
    mlens_ref[...] = ml


def _sc_body(keys_hbm, rowmax_hbm, qpred_hbm, fpred_hbm, fa0_hbm, fa1_hbm,
             vout_hbm, mpred_hbm, ma0_hbm, ma1_hbm,
             rowmax_v, lvl2_v, rowbuf, idxbuf, keybuf, obuf,
             gpred, ga0, ga1, qv_v, dirty_v, dmap_v, sem):
    q = lax.axis_index("s") * 2 + lax.axis_index("c")
    iota16 = jax.lax.broadcasted_iota(jnp.int32, (16,), 0)

    pltpu.sync_copy(rowmax_hbm.at[q], rowmax_v)
    pltpu.sync_copy(qpred_hbm, qv_v)

    l3v = jnp.full((16,), IMIN, jnp.int32)
    for k in range(16):
        acc = jnp.full((16,), IMIN, jnp.int32)
        for o in range(16):
            v = rowmax_v[pl.ds((k * 16 + o) * 16, 16)]
            acc = jnp.where(iota16 == o, jnp.max(v), acc)
        lvl2_v[pl.ds(k * 16, 16)] = acc
        l3v = jnp.where(iota16 == k, jnp.max(acc), l3v)

    for j in range(4):
        dmap_v[pl.ds(j * 16, 16)] = jnp.full((16,), -1, jnp.int32)

    def ext_body(t, carry):
        l3v, nslots = carry
        m = jnp.max(l3v)
        kstar = jnp.min(jnp.where(l3v == m, iota16, 16))
        lv2k = lvl2_v[pl.ds(kstar * 16, 16)]
        tstar = jnp.min(jnp.where(lv2k == m, iota16, 16))
        cb = kstar * 256 + tstar * 16
        ch = rowmax_v[pl.ds(cb, 16)]
        lstar = jnp.min(jnp.where(ch == m, iota16, 16))
        r = cb + lstar

        slotvec = jnp.full((16,), BIGI, jnp.int32)
        for j in range(4):
            mv = dmap_v[pl.ds(j * 16, 16)]
            slotvec = jnp.minimum(
                slotvec, jnp.where(mv == r, iota16 + j * 16, BIGI))
        found_slot = jnp.min(slotvec)
        hit = found_slot < BIGI
        slot = jnp.where(hit, found_slot, nslots)
        hitv = iota16 * 0 + jnp.where(hit, 1, 0) > 0

        pltpu.sync_copy(keys_hbm.at[q, r], rowbuf)
        jb = (nslots // 16) * 16
        mv = dmap_v[pl.ds(jb, 16)]
        dmap_v[pl.ds(jb, 16)] = jnp.where(
            (iota16 == nslots - jb) & jnp.logical_not(hitv), r, mv)
        nslots = jnp.where(hit, nslots, nslots + 1)

        rowvs = []
        minvec = jnp.full((16,), BIGI, jnp.int32)
        for j in range(8):
            fresh = rowbuf[pl.ds(j * 16, 16)]
            cached = dirty_v[pl.ds(slot * 128 + j * 16, 16)]
            v = jnp.where(hitv, cached, fresh)
            rowvs.append(v)
            minvec = jnp.minimum(
                minvec, jnp.where(v == m, iota16 + j * 16, BIGI))
        c = jnp.min(minvec)
        nmvec = jnp.full((16,), IMIN, jnp.int32)
        for j in range(8):
            v = jnp.where(iota16 + j * 16 == c, IMIN, rowvs[j])
            dirty_v[pl.ds(slot * 128 + j * 16, 16)] = v
            nmvec = jnp.maximum(nmvec, v)
        nm = jnp.max(nmvec)
        gidx = r * 128 + c

        tb = (t // 16) * 16
        iv = idxbuf[pl.ds(tb, 16)]
        idxbuf[pl.ds(tb, 16)] = jnp.where(iota16 == t - tb, gidx, iv)
        kv = keybuf[pl.ds(tb, 16)]
        keybuf[pl.ds(tb, 16)] = jnp.where(iota16 == t - tb, m, kv)
        rowmax_v[pl.ds(cb, 16)] = jnp.where(iota16 == lstar, nm, ch)
        nm2 = jnp.max(jnp.where(iota16 == lstar, nm, ch))
        lvl2_v[pl.ds(kstar * 16, 16)] = jnp.where(
            iota16 == tstar, nm2, lv2k)
        lv2k2 = jnp.where(iota16 == tstar, nm2, lv2k)
        return (jnp.where(iota16 == kstar, jnp.max(lv2k2), l3v),
                nslots)

    lax.fori_loop(0, KF, ext_body, (l3v, jnp.int32(0)))

    pltpu.async_copy(fpred_hbm.at[idxbuf], gpred, sem).wait()
    pltpu.async_copy(fa0_hbm.at[idxbuf], ga0, sem).wait()
    pltpu.async_copy(fa1_hbm.at[idxbuf], ga1, sem).wait()

    qb = (q // 16) * 16
    qchunk = qv_v[pl.ds(qb, 16)]
    qsel = jnp.where(iota16 == q - qb, qchunk, 0)
    act = iota16 * 0 + jnp.max(qsel) != 0
    for j in range(4):
        sl = pl.ds(j * 16, 16)
        kb = keybuf[sl]
        valid = (kb > KEY_NEG1E8) & act
        fb = plsc.bitcast(jnp.where(kb >= 0, kb, kb ^ MASK31), jnp.float32)
        obuf[sl] = jnp.where(valid, fb, jnp.float32(0.0))
        gpred[sl] = jnp.where(valid, gpred[sl], 0)
        ga0[sl] = jnp.where(valid, ga0[sl], 0)
        ga1[sl] = jnp.where(valid, ga1[sl], 0)

    pltpu.sync_copy(obuf, vout_hbm.at[q])
    pltpu.sync_copy(gpred, mpred_hbm.at[q])
    pltpu.sync_copy(ga0, ma0_hbm.at[q])
    pltpu.sync_copy(ga1, ma1_hbm.at[q])


def _pred_body(vout_ref, qpred_ref, pred_ref):
    w = jnp.sum(vout_ref[...], axis=1, keepdims=True)
    for ci in range(2):
        classes = _iota((1, 128), 1) + ci * 128
        oh = qpred_ref[...] == classes
        pred_ref[0:1, pl.ds(ci * 128, 128)] = jnp.sum(
            jnp.where(oh, w, jnp.float32(0.0)), axis=0).reshape(1, 128)


def kernel(proof_goals, facts_idx, rules_heads_idx, rules_bodies_idx,
           rule_lens, fact_weights):
    B, S, G, _ = proof_goals.shape
    F = facts_idx.shape[0]
    R = rules_bodies_idx.shape[0]
    BMAX = rules_bodies_idx.shape[1]
    nq = B * S
    nrow = pl.cdiv(F, 128)
    nrow = ((nrow + 127) // 128) * 128
    fp = nrow * 128

    qarr = proof_goals[:, :, 0, :].reshape(nq, 3)
    pad = fp - F
    fT = jnp.pad(facts_idx.T, ((0, 0), (0, pad)))
    fpred = fT[0].reshape(nrow, 128)
    fa0 = fT[1].reshape(nrow, 128)
    fa1 = fT[2].reshape(nrow, 128)
    wts = jnp.pad(fact_weights, (0, pad)).reshape(nrow, 128)
    rhead = rules_heads_idx[:, 0].reshape(1, R)
    bodies = rules_bodies_idx.reshape(R, 3 * BMAX)
    rlens = rule_lens.reshape(1, R)

    ngroups = nq // GQ
    smem = pl.BlockSpec(memory_space=pltpu.SMEM)
    vfull = pl.BlockSpec(memory_space=pltpu.VMEM)

    keys, rowmax, mbodies, mlens = pl.pallas_call(
        functools.partial(_pass1_body, nrow, R, BMAX),
        grid=(ngroups,),
        out_shape=(
            jax.ShapeDtypeStruct((nq, nrow, 128), jnp.int32),
            jax.ShapeDtypeStruct((nq, NROWPAD), jnp.int32),
            jax.ShapeDtypeStruct((nq, KR, 3 * BMAX), jnp.int32),
            jax.ShapeDtypeStruct((nq, KR), jnp.int32),
        ),
        in_specs=[
            pl.BlockSpec((nq, 3), lambda i: (0, 0), memory_space=pltpu.SMEM),
            pl.BlockSpec((1, R), lambda i: (0, 0), memory_space=pltpu.SMEM),
            pl.BlockSpec((nrow, 128), lambda i: (0, 0)),
            pl.BlockSpec((nrow, 128), lambda i: (0, 0)),
            pl.BlockSpec((nrow, 128), lambda i: (0, 0)),
            pl.BlockSpec((nrow, 128), lambda i: (0, 0)),
            pl.BlockSpec((1, R), lambda i: (0, 0)),
            pl.BlockSpec((R, 3 * BMAX), lambda i: (0, 0)),
        ],
        out_specs=(
            pl.BlockSpec((GQ, nrow, 128), lambda i: (i, 0, 0)),
            pl.BlockSpec((GQ, NROWPAD), lambda i: (i, 0)),
            pl.BlockSpec((GQ, KR, 3 * BMAX), lambda i: (i, 0, 0)),
            pl.BlockSpec((GQ, KR), lambda i: (i, 0)),
        ),
        scratch_shapes=[pltpu.VMEM((GQ, R), jnp.float32)],
    )(qarr, rlens, fpred, fa0, fa1, wts, rhead, bodies)

    mesh = plsc.VectorSubcoreMesh(core_axis_name="c", subcore_axis_name="s")
    sc_out = (
        jax.ShapeDtypeStruct((nq, KF), jnp.float32),
        jax.ShapeDtypeStruct((nq, KF), jnp.int32),
        jax.ShapeDtypeStruct((nq, KF), jnp.int32),
        jax.ShapeDtypeStruct((nq, KF), jnp.int32),
    )
    sc_scratch = [
        pltpu.VMEM((NROWPAD,), jnp.int32),
        pltpu.VMEM((256,), jnp.int32),
        pltpu.VMEM((128,), jnp.int32),
        pltpu.VMEM((KF,), jnp.int32),
        pltpu.VMEM((KF,), jnp.int32),
        pltpu.VMEM((KF,), jnp.float32),
        pltpu.VMEM((KF,), jnp.int32),
        pltpu.VMEM((KF,), jnp.int32),
        pltpu.VMEM((KF,), jnp.int32),
        pltpu.VMEM((nq,), jnp.int32),
        pltpu.VMEM((KF * 128,), jnp.int32),
        pltpu.VMEM((KF,), jnp.int32),
        pltpu.SemaphoreType.DMA,
    ]
    qpredv = qarr[:, 0]
    vout, mpred, ma0, ma1 = pl.kernel(
        _sc_body, mesh=mesh, out_type=sc_out, scratch_types=sc_scratch,
        compiler_params=pltpu.CompilerParams(needs_layout_passes=False),
    )(keys, rowmax, qpredv, fpred.reshape(fp), fa0.reshape(fp),
      fa1.reshape(fp))

    pred = pl.pallas_call(
        _pred_body,
        out_shape=jax.ShapeDtypeStruct((1, 256), jnp.float32),
        in_specs=[vfull, vfull],
        out_specs=vfull,
    )(vout, qpredv.reshape(nq, 1))

    matched_facts = jnp.stack([mpred, ma0, ma1], axis=-1).reshape(B, S, KF, 3)
    return (vout.reshape(B, S, KF),
            matched_facts,
            mbodies.reshape(B, S, KR, BMAX, 3),
            mlens.reshape(B, S, KR),
            pred[0, :201])
